# baseline (speedup 1.0000x reference)
import functools

import jax
import jax.numpy as jnp
from jax import lax
from jax.experimental import pallas as pl
from jax.experimental.pallas import tpu as pltpu
from jax.experimental.pallas import tpu_sc as plsc

D = 1024
N = 2048
E = 8
CAP = 256
H = 4096
EPAD = 128
PADROW = N
SENT = 4095

NC, NS, L = 2, 16, 16
NW = NC * NS
TPW = N // NW
CH = TPW // 2


def _router_body(x_ref, wh_ref, f0_ref, f1_ref, w0_ref, w1_ref):
    x = x_ref[...]
    wh = wh_ref[...]
    logits = jnp.dot(x, wh, preferred_element_type=jnp.float32)
    eidx = lax.broadcasted_iota(jnp.int32, (N, EPAD), 1)
    neg = jnp.float32(-1e30)
    logits = jnp.where(eidx < E, logits, neg)

    big = jnp.int32(2**30)
    m1 = jnp.max(logits, axis=1, keepdims=True)
    i1 = jnp.min(jnp.where(logits == m1, eidx, big), axis=1, keepdims=True)
    l2 = jnp.where(eidx == i1, neg, logits)
    m2 = jnp.max(l2, axis=1, keepdims=True)
    i2 = jnp.min(jnp.where(l2 == m2, eidx, big), axis=1, keepdims=True)

    ed = jnp.exp(m2 - m1)
    g0 = 1.0 / (1.0 + ed)
    g1 = ed / (1.0 + ed)

    ohc = ((eidx == i1) | (eidx == i2 + E)).astype(jnp.float32)
    cum = ohc
    s = 1
    while s < N:
        shifted = jnp.concatenate(
            [jnp.zeros((s, EPAD), jnp.float32), cum[: N - s, :]], axis=0)
        cum = cum + shifted
        s *= 2
    oh0f = (eidx == i1).astype(jnp.float32)
    oh1f = (eidx == i2 + E).astype(jnp.float32)
    p0 = jnp.sum(cum * oh0f, axis=1, keepdims=True)
    p1 = jnp.sum(cum * oh1f, axis=1, keepdims=True)
    s0 = p0.astype(jnp.int32) - 1
    s1 = p1.astype(jnp.int32) - 1
    v0 = s0 < CAP
    v1 = s1 < CAP
    f0_ref[...] = jnp.where(v0, i1 * CAP + s0, SENT)
    f1_ref[...] = jnp.where(v1, i2 * CAP + s1, SENT)
    w0_ref[...] = jnp.where(v0, g0, 0.0)
    w1_ref[...] = jnp.where(v1, g1, 0.0)


_router = pl.pallas_call(
    _router_body,
    out_shape=(
        jax.ShapeDtypeStruct((N, 1), jnp.int32),
        jax.ShapeDtypeStruct((N, 1), jnp.int32),
        jax.ShapeDtypeStruct((N, 1), jnp.float32),
        jax.ShapeDtypeStruct((N, 1), jnp.float32),
    ),
)


HBLK = 512
NHBLK = H // HBLK


def _moe_body(f0r_ref, f1r_ref, xb_ref, w1_ref, w2_ref, out_ref, ei_s):
    e = pl.program_id(0)
    h = pl.program_id(1)

    @pl.when(h == 0)
    def _():
        slot = e * CAP + lax.broadcasted_iota(jnp.int32, (CAP, N), 0)
        oh = ((f0r_ref[...] == slot) | (f1r_ref[...] == slot))
        ei = jnp.dot(oh.astype(jnp.bfloat16), xb_ref[...],
                     preferred_element_type=jnp.float32)
        ei_s[...] = ei.astype(jnp.bfloat16)

    act = jnp.dot(ei_s[...], w1_ref[0].astype(jnp.bfloat16),
                  preferred_element_type=jnp.float32)
    act = jax.nn.gelu(act).astype(jnp.bfloat16)
    part = jnp.dot(act, w2_ref[0].astype(jnp.bfloat16),
                   preferred_element_type=jnp.float32)

    @pl.when(h == 0)
    def _():
        out_ref[...] = part

    @pl.when(h != 0)
    def _():
        out_ref[...] += part


_moe = pl.pallas_call(
    _moe_body,
    grid=(E, NHBLK),
    in_specs=[
        pl.BlockSpec((1, N), lambda e, h: (0, 0)),
        pl.BlockSpec((1, N), lambda e, h: (0, 0)),
        pl.BlockSpec((N, D), lambda e, h: (0, 0)),
        pl.BlockSpec((1, D, HBLK), lambda e, h: (e, 0, h)),
        pl.BlockSpec((1, HBLK, D), lambda e, h: (e, h, 0)),
    ],
    out_specs=pl.BlockSpec((CAP, D), lambda e, h: (e, 0)),
    out_shape=jax.ShapeDtypeStruct((E * CAP, D), jnp.float32),
    scratch_shapes=[pltpu.VMEM((CAP, D), jnp.bfloat16)],
    compiler_params=pltpu.CompilerParams(
        dimension_semantics=("parallel", "arbitrary"),
    ),
)


@functools.cache
def _sc_kernels():
    mesh = plsc.VectorSubcoreMesh(
        core_axis_name="c", subcore_axis_name="s",
        num_cores=NC, num_subcores=NS,
    )

    @functools.partial(
        pl.kernel,
        out_type=jax.ShapeDtypeStruct((N, D), jnp.float32),
        mesh=mesh,
        compiler_params=pltpu.CompilerParams(needs_layout_passes=False),
        scratch_types=[
            pltpu.VMEM((TPW,), jnp.int32),
            pltpu.VMEM((TPW,), jnp.int32),
            pltpu.VMEM((TPW,), jnp.float32),
            pltpu.VMEM((TPW,), jnp.float32),
            pltpu.VMEM((CH, D), jnp.float32),
            pltpu.VMEM((CH, D), jnp.float32),
            pltpu.SemaphoreType.DMA,
            pltpu.SemaphoreType.DMA,
        ],
    )
    def combine(eo_hbm, f0_hbm, f1_hbm, w0_hbm, w1_hbm, out_hbm,
                i0_v, i1_v, w0_v, w1_v, ra_v, rb_v, sem, sem_b):
        wid = lax.axis_index("s") * NC + lax.axis_index("c")
        lo = wid * TPW

        pltpu.sync_copy(f0_hbm.at[pl.ds(lo, TPW)], i0_v)
        pltpu.sync_copy(f1_hbm.at[pl.ds(lo, TPW)], i1_v)
        pltpu.sync_copy(w0_hbm.at[pl.ds(lo, TPW)], w0_v)
        pltpu.sync_copy(w1_hbm.at[pl.ds(lo, TPW)], w1_v)

        lim = jnp.full((L,), E * CAP - 1, jnp.int32)
        for j in range(TPW // L):
            sl = pl.ds(j * L, L)
            i0_v[sl] = jnp.minimum(i0_v[sl], lim)
            i1_v[sl] = jnp.minimum(i1_v[sl], lim)

        for ci in range(2):
            ga = pltpu.async_copy(
                eo_hbm.at[i0_v.at[pl.ds(ci * CH, CH)]], ra_v, sem)
            gb = pltpu.async_copy(
                eo_hbm.at[i1_v.at[pl.ds(ci * CH, CH)]], rb_v, sem_b)
            ga.wait()
            gb.wait()

            def wsum_row(rr, _):
                bidx = jnp.full((L,), ci * CH, jnp.int32) + rr
                w0s = plsc.load_gather(w0_v, [bidx])
                w1s = plsc.load_gather(w1_v, [bidx])
                for j in range(D // L):
                    sl = pl.ds(j * L, L)
                    ra_v[rr, sl] = ra_v[rr, sl] * w0s + rb_v[rr, sl] * w1s
                return 0

            lax.fori_loop(0, CH, wsum_row, 0)
            pltpu.sync_copy(ra_v, out_hbm.at[pl.ds(lo + ci * CH, CH)])

    return combine


def kernel(x, W_hash, expert_w1, expert_w2):
    combine = _sc_kernels()

    x_flat = x.reshape(N, D)
    xb = x_flat.astype(jnp.bfloat16)
    whp = jnp.pad(W_hash, ((0, 0), (0, EPAD - E)))

    f0, f1, w0, w1 = _router(x_flat, whp)
    f0r = f0.reshape(1, N)
    f1r = f1.reshape(1, N)

    eo = _moe(f0r, f1r, xb, expert_w1, expert_w2)
    out = combine(eo, f0.reshape(N), f1.reshape(N), w0.reshape(N), w1.reshape(N))
    return out.reshape(1, N, D)


# speedup vs baseline: 1.0323x; 1.0323x over previous
import functools

import jax
import jax.numpy as jnp
from jax import lax
from jax.experimental import pallas as pl
from jax.experimental.pallas import tpu as pltpu
from jax.experimental.pallas import tpu_sc as plsc

D = 1024
N = 2048
E = 8
CAP = 256
H = 4096
EPAD = 128
PADROW = N
SENT = 4095

NC, NS, L = 2, 16, 16
NW = NC * NS
TPW = N // NW
CH = TPW // 2


def _router_body(x_ref, wh_ref, f0_ref, f1_ref, w0_ref, w1_ref, xb_ref):
    x = x_ref[...]
    xb_ref[...] = x.astype(jnp.bfloat16)
    wh = wh_ref[...]
    logits = jnp.dot(x, wh, preferred_element_type=jnp.float32)
    eidx = lax.broadcasted_iota(jnp.int32, (N, EPAD), 1)
    neg = jnp.float32(-1e30)
    logits = jnp.where(eidx < E, logits, neg)

    big = jnp.int32(2**30)
    m1 = jnp.max(logits, axis=1, keepdims=True)
    i1 = jnp.min(jnp.where(logits == m1, eidx, big), axis=1, keepdims=True)
    l2 = jnp.where(eidx == i1, neg, logits)
    m2 = jnp.max(l2, axis=1, keepdims=True)
    i2 = jnp.min(jnp.where(l2 == m2, eidx, big), axis=1, keepdims=True)

    ed = jnp.exp(m2 - m1)
    g0 = 1.0 / (1.0 + ed)
    g1 = ed / (1.0 + ed)

    ohc = ((eidx == i1) | (eidx == i2 + E)).astype(jnp.float32)
    cum = ohc
    s = 1
    while s < N:
        shifted = jnp.concatenate(
            [jnp.zeros((s, EPAD), jnp.float32), cum[: N - s, :]], axis=0)
        cum = cum + shifted
        s *= 2
    oh0f = (eidx == i1).astype(jnp.float32)
    oh1f = (eidx == i2 + E).astype(jnp.float32)
    p0 = jnp.sum(cum * oh0f, axis=1, keepdims=True)
    p1 = jnp.sum(cum * oh1f, axis=1, keepdims=True)
    s0 = p0.astype(jnp.int32) - 1
    s1 = p1.astype(jnp.int32) - 1
    v0 = s0 < CAP
    v1 = s1 < CAP
    f0_ref[...] = jnp.where(v0, i1 * CAP + s0, SENT)
    f1_ref[...] = jnp.where(v1, i2 * CAP + s1, SENT)
    w0_ref[...] = jnp.where(v0, g0, 0.0)
    w1_ref[...] = jnp.where(v1, g1, 0.0)


_router = pl.pallas_call(
    _router_body,
    out_shape=(
        jax.ShapeDtypeStruct((N, 1), jnp.int32),
        jax.ShapeDtypeStruct((N, 1), jnp.int32),
        jax.ShapeDtypeStruct((N, 1), jnp.float32),
        jax.ShapeDtypeStruct((N, 1), jnp.float32),
        jax.ShapeDtypeStruct((N, D), jnp.bfloat16),
    ),
)


HBLK = 512
NHBLK = H // HBLK


def _moe_body(f0r_ref, f1r_ref, xb_ref, w1_ref, w2_ref, out_ref, ei_s):
    e = pl.program_id(0)
    h = pl.program_id(1)

    @pl.when(h == 0)
    def _():
        slot = e * CAP + lax.broadcasted_iota(jnp.int32, (CAP, N), 0)
        oh = ((f0r_ref[...] == slot) | (f1r_ref[...] == slot))
        ei = jnp.dot(oh.astype(jnp.bfloat16), xb_ref[...],
                     preferred_element_type=jnp.float32)
        ei_s[...] = ei.astype(jnp.bfloat16)

    act = jnp.dot(ei_s[...], w1_ref[0].astype(jnp.bfloat16),
                  preferred_element_type=jnp.float32)
    act = jax.nn.gelu(act).astype(jnp.bfloat16)
    part = jnp.dot(act, w2_ref[0].astype(jnp.bfloat16),
                   preferred_element_type=jnp.float32)

    @pl.when(h == 0)
    def _():
        out_ref[...] = part

    @pl.when(h != 0)
    def _():
        out_ref[...] += part


_moe = pl.pallas_call(
    _moe_body,
    grid=(E, NHBLK),
    in_specs=[
        pl.BlockSpec((1, N), lambda e, h: (0, 0)),
        pl.BlockSpec((1, N), lambda e, h: (0, 0)),
        pl.BlockSpec((N, D), lambda e, h: (0, 0)),
        pl.BlockSpec((1, D, HBLK), lambda e, h: (e, 0, h)),
        pl.BlockSpec((1, HBLK, D), lambda e, h: (e, h, 0)),
    ],
    out_specs=pl.BlockSpec((CAP, D), lambda e, h: (e, 0)),
    out_shape=jax.ShapeDtypeStruct((E * CAP, D), jnp.float32),
    scratch_shapes=[pltpu.VMEM((CAP, D), jnp.bfloat16)],
    compiler_params=pltpu.CompilerParams(
        dimension_semantics=("parallel", "arbitrary"),
    ),
)


@functools.cache
def _sc_kernels():
    mesh = plsc.VectorSubcoreMesh(
        core_axis_name="c", subcore_axis_name="s",
        num_cores=NC, num_subcores=NS,
    )

    CC = 16
    NCH = TPW // CC

    @functools.partial(
        pl.kernel,
        out_type=jax.ShapeDtypeStruct((N, D), jnp.float32),
        mesh=mesh,
        compiler_params=pltpu.CompilerParams(needs_layout_passes=False),
        scratch_types=[
            pltpu.VMEM((TPW,), jnp.int32),
            pltpu.VMEM((TPW,), jnp.int32),
            pltpu.VMEM((TPW,), jnp.float32),
            pltpu.VMEM((TPW,), jnp.float32),
            pltpu.VMEM((2, CC, D), jnp.float32),
            pltpu.VMEM((2, CC, D), jnp.float32),
            pltpu.SemaphoreType.DMA,
            pltpu.SemaphoreType.DMA,
            pltpu.SemaphoreType.DMA,
        ],
    )
    def combine(eo_hbm, f0_hbm, f1_hbm, w0_hbm, w1_hbm, out_hbm,
                i0_v, i1_v, w0_v, w1_v, ra_v, rb_v, sem_a, sem_b, sem_w):
        wid = lax.axis_index("s") * NC + lax.axis_index("c")
        lo = wid * TPW

        pltpu.sync_copy(f0_hbm.at[pl.ds(lo, TPW)], i0_v)
        pltpu.sync_copy(f1_hbm.at[pl.ds(lo, TPW)], i1_v)
        pltpu.sync_copy(w0_hbm.at[pl.ds(lo, TPW)], w0_v)
        pltpu.sync_copy(w1_hbm.at[pl.ds(lo, TPW)], w1_v)

        lim = jnp.full((L,), E * CAP - 1, jnp.int32)
        for j in range(TPW // L):
            sl = pl.ds(j * L, L)
            i0_v[sl] = jnp.minimum(i0_v[sl], lim)
            i1_v[sl] = jnp.minimum(i1_v[sl], lim)

        def gathers(ci):
            b = ci % 2
            ga = pltpu.async_copy(
                eo_hbm.at[i0_v.at[pl.ds(ci * CC, CC)]], ra_v.at[b], sem_a)
            gb = pltpu.async_copy(
                eo_hbm.at[i1_v.at[pl.ds(ci * CC, CC)]], rb_v.at[b], sem_b)
            return ga, gb

        inflight = gathers(0)
        writes = [None, None]
        for ci in range(NCH):
            b = ci % 2
            ga, gb = inflight
            ga.wait()
            gb.wait()
            if ci + 1 < NCH:
                ob = (ci + 1) % 2
                if writes[ob] is not None:
                    writes[ob].wait()
                    writes[ob] = None
                inflight = gathers(ci + 1)

            def wsum_row(rr, _):
                bidx = jnp.full((L,), ci * CC, jnp.int32) + rr
                w0s = plsc.load_gather(w0_v, [bidx])
                w1s = plsc.load_gather(w1_v, [bidx])
                for j in range(D // L):
                    sl = pl.ds(j * L, L)
                    ra_v[b, rr, sl] = (
                        ra_v[b, rr, sl] * w0s + rb_v[b, rr, sl] * w1s)
                return 0

            lax.fori_loop(0, CC, wsum_row, 0)
            writes[b] = pltpu.async_copy(
                ra_v.at[b], out_hbm.at[pl.ds(lo + ci * CC, CC)], sem_w)
        for wr in writes:
            if wr is not None:
                wr.wait()

    return combine


def kernel(x, W_hash, expert_w1, expert_w2):
    combine = _sc_kernels()

    x_flat = x.reshape(N, D)
    whp = jnp.pad(W_hash, ((0, 0), (0, EPAD - E)))

    f0, f1, w0, w1, xb = _router(x_flat, whp)
    f0r = f0.reshape(1, N)
    f1r = f1.reshape(1, N)

    eo = _moe(f0r, f1r, xb, expert_w1, expert_w2)
    out = combine(eo, f0.reshape(N), f1.reshape(N), w0.reshape(N), w1.reshape(N))
    return out.reshape(1, N, D)


# speedup vs baseline: 1.1587x; 1.1224x over previous
import functools

import jax
import jax.numpy as jnp
from jax import lax
from jax.experimental import pallas as pl
from jax.experimental.pallas import tpu as pltpu
from jax.experimental.pallas import tpu_sc as plsc

D = 1024
N = 2048
E = 8
CAP = 256
H = 4096
EPAD = 128
PADROW = N
SENT = 4095

NC, NS, L = 2, 16, 16
NW = NC * NS
TPW = N // NW
CH = TPW // 2


def _router_body(x_ref, wh_ref, f0_ref, f1_ref, w0_ref, w1_ref, xb_ref):
    x = x_ref[...]
    xb_ref[...] = x.astype(jnp.bfloat16)
    wh = wh_ref[...]
    logits = jnp.dot(x, wh, preferred_element_type=jnp.float32)
    eidx = lax.broadcasted_iota(jnp.int32, (N, EPAD), 1)
    neg = jnp.float32(-1e30)
    logits = jnp.where(eidx < E, logits, neg)

    big = jnp.int32(2**30)
    m1 = jnp.max(logits, axis=1, keepdims=True)
    i1 = jnp.min(jnp.where(logits == m1, eidx, big), axis=1, keepdims=True)
    l2 = jnp.where(eidx == i1, neg, logits)
    m2 = jnp.max(l2, axis=1, keepdims=True)
    i2 = jnp.min(jnp.where(l2 == m2, eidx, big), axis=1, keepdims=True)

    ed = jnp.exp(m2 - m1)
    g0 = 1.0 / (1.0 + ed)
    g1 = ed / (1.0 + ed)

    ohc = ((eidx == i1) | (eidx == i2 + E)).astype(jnp.float32)
    cum = ohc
    s = 1
    while s < N:
        shifted = jnp.concatenate(
            [jnp.zeros((s, EPAD), jnp.float32), cum[: N - s, :]], axis=0)
        cum = cum + shifted
        s *= 2
    oh0f = (eidx == i1).astype(jnp.float32)
    oh1f = (eidx == i2 + E).astype(jnp.float32)
    p0 = jnp.sum(cum * oh0f, axis=1, keepdims=True)
    p1 = jnp.sum(cum * oh1f, axis=1, keepdims=True)
    s0 = p0.astype(jnp.int32) - 1
    s1 = p1.astype(jnp.int32) - 1
    v0 = s0 < CAP
    v1 = s1 < CAP
    f0_ref[...] = jnp.where(v0, i1 * CAP + s0, SENT)
    f1_ref[...] = jnp.where(v1, i2 * CAP + s1, SENT)
    w0_ref[...] = jnp.where(v0, g0, 0.0)
    w1_ref[...] = jnp.where(v1, g1, 0.0)


_router = pl.pallas_call(
    _router_body,
    out_shape=(
        jax.ShapeDtypeStruct((N, 1), jnp.int32),
        jax.ShapeDtypeStruct((N, 1), jnp.int32),
        jax.ShapeDtypeStruct((N, 1), jnp.float32),
        jax.ShapeDtypeStruct((N, 1), jnp.float32),
        jax.ShapeDtypeStruct((N, D), jnp.bfloat16),
    ),
)


HBLK = 1024
NHBLK = H // HBLK


def _moe_body(f0r_ref, f1r_ref, xb_ref, w1_ref, w2_ref, out_ref, ei_s):
    e = pl.program_id(0)
    h = pl.program_id(1)

    @pl.when(h == 0)
    def _():
        slot = e * CAP + lax.broadcasted_iota(jnp.int32, (CAP, N), 0)
        oh = ((f0r_ref[...] == slot) | (f1r_ref[...] == slot))
        ei = jnp.dot(oh.astype(jnp.bfloat16), xb_ref[...],
                     preferred_element_type=jnp.float32)
        ei_s[...] = ei.astype(jnp.bfloat16)

    act = jnp.dot(ei_s[...], w1_ref[0].astype(jnp.bfloat16),
                  preferred_element_type=jnp.float32)
    act = jax.nn.gelu(act).astype(jnp.bfloat16)
    part = jnp.dot(act, w2_ref[0].astype(jnp.bfloat16),
                   preferred_element_type=jnp.float32)

    @pl.when(h == 0)
    def _():
        out_ref[...] = part

    @pl.when(h != 0)
    def _():
        out_ref[...] += part


_moe = pl.pallas_call(
    _moe_body,
    grid=(E, NHBLK),
    in_specs=[
        pl.BlockSpec((1, N), lambda e, h: (0, 0)),
        pl.BlockSpec((1, N), lambda e, h: (0, 0)),
        pl.BlockSpec((N, D), lambda e, h: (0, 0)),
        pl.BlockSpec((1, D, HBLK), lambda e, h: (e, 0, h)),
        pl.BlockSpec((1, HBLK, D), lambda e, h: (e, h, 0)),
    ],
    out_specs=pl.BlockSpec((CAP, D), lambda e, h: (e, 0)),
    out_shape=jax.ShapeDtypeStruct((E * CAP, D), jnp.float32),
    scratch_shapes=[pltpu.VMEM((CAP, D), jnp.bfloat16)],
    compiler_params=pltpu.CompilerParams(
        dimension_semantics=("parallel", "arbitrary"),
    ),
)


@functools.cache
def _sc_kernels():
    mesh = plsc.VectorSubcoreMesh(
        core_axis_name="c", subcore_axis_name="s",
        num_cores=NC, num_subcores=NS,
    )

    CC = 16
    NCH = TPW // CC

    @functools.partial(
        pl.kernel,
        out_type=jax.ShapeDtypeStruct((N, D), jnp.float32),
        mesh=mesh,
        compiler_params=pltpu.CompilerParams(needs_layout_passes=False),
        scratch_types=[
            pltpu.VMEM((TPW,), jnp.int32),
            pltpu.VMEM((TPW,), jnp.int32),
            pltpu.VMEM((TPW,), jnp.float32),
            pltpu.VMEM((TPW,), jnp.float32),
            pltpu.VMEM((2, CC, D), jnp.float32),
            pltpu.VMEM((2, CC, D), jnp.float32),
            pltpu.SemaphoreType.DMA,
            pltpu.SemaphoreType.DMA,
            pltpu.SemaphoreType.DMA,
        ],
    )
    def combine(eo_hbm, f0_hbm, f1_hbm, w0_hbm, w1_hbm, out_hbm,
                i0_v, i1_v, w0_v, w1_v, ra_v, rb_v, sem_a, sem_b, sem_w):
        wid = lax.axis_index("s") * NC + lax.axis_index("c")
        lo = wid * TPW

        pltpu.sync_copy(f0_hbm.at[pl.ds(lo, TPW)], i0_v)
        pltpu.sync_copy(f1_hbm.at[pl.ds(lo, TPW)], i1_v)
        pltpu.sync_copy(w0_hbm.at[pl.ds(lo, TPW)], w0_v)
        pltpu.sync_copy(w1_hbm.at[pl.ds(lo, TPW)], w1_v)

        lim = jnp.full((L,), E * CAP - 1, jnp.int32)
        for j in range(TPW // L):
            sl = pl.ds(j * L, L)
            i0_v[sl] = jnp.minimum(i0_v[sl], lim)
            i1_v[sl] = jnp.minimum(i1_v[sl], lim)

        def gathers(ci):
            b = ci % 2
            ga = pltpu.async_copy(
                eo_hbm.at[i0_v.at[pl.ds(ci * CC, CC)]], ra_v.at[b], sem_a)
            gb = pltpu.async_copy(
                eo_hbm.at[i1_v.at[pl.ds(ci * CC, CC)]], rb_v.at[b], sem_b)
            return ga, gb

        inflight = gathers(0)
        writes = [None, None]
        for ci in range(NCH):
            b = ci % 2
            ga, gb = inflight
            ga.wait()
            gb.wait()
            if ci + 1 < NCH:
                ob = (ci + 1) % 2
                if writes[ob] is not None:
                    writes[ob].wait()
                    writes[ob] = None
                inflight = gathers(ci + 1)

            def wsum_row(rr, _):
                bidx = jnp.full((L,), ci * CC, jnp.int32) + rr
                w0s = plsc.load_gather(w0_v, [bidx])
                w1s = plsc.load_gather(w1_v, [bidx])
                for j in range(D // L):
                    sl = pl.ds(j * L, L)
                    ra_v[b, rr, sl] = (
                        ra_v[b, rr, sl] * w0s + rb_v[b, rr, sl] * w1s)
                return 0

            lax.fori_loop(0, CC, wsum_row, 0)
            writes[b] = pltpu.async_copy(
                ra_v.at[b], out_hbm.at[pl.ds(lo + ci * CC, CC)], sem_w)
        for wr in writes:
            if wr is not None:
                wr.wait()

    return combine


def kernel(x, W_hash, expert_w1, expert_w2):
    combine = _sc_kernels()

    x_flat = x.reshape(N, D)
    whp = jnp.pad(W_hash, ((0, 0), (0, EPAD - E)))

    f0, f1, w0, w1, xb = _router(x_flat, whp)
    f0r = f0.reshape(1, N)
    f1r = f1.reshape(1, N)

    eo = _moe(f0r, f1r, xb, expert_w1, expert_w2)
    out = combine(eo, f0.reshape(N), f1.reshape(N), w0.reshape(N), w1.reshape(N))
    return out.reshape(1, N, D)


# speedup vs baseline: 1.2275x; 1.0594x over previous
import functools

import jax
import jax.numpy as jnp
from jax import lax
from jax.experimental import pallas as pl
from jax.experimental.pallas import tpu as pltpu
from jax.experimental.pallas import tpu_sc as plsc

D = 1024
N = 2048
E = 8
CAP = 256
H = 4096
EPAD = 128
PADROW = N
SENT = 4095

NC, NS, L = 2, 16, 16
NW = NC * NS
TPW = N // NW
CH = TPW // 2


def _router_body(x_ref, wh_ref, f0_ref, f1_ref, w0_ref, w1_ref, xb_ref):
    x = x_ref[...]
    xb_ref[...] = x.astype(jnp.bfloat16)
    wh = wh_ref[...]
    logits = jnp.dot(x, wh, preferred_element_type=jnp.float32)
    eidx = lax.broadcasted_iota(jnp.int32, (N, EPAD), 1)
    neg = jnp.float32(-1e30)
    logits = jnp.where(eidx < E, logits, neg)

    big = jnp.int32(2**30)
    m1 = jnp.max(logits, axis=1, keepdims=True)
    i1 = jnp.min(jnp.where(logits == m1, eidx, big), axis=1, keepdims=True)
    l2 = jnp.where(eidx == i1, neg, logits)
    m2 = jnp.max(l2, axis=1, keepdims=True)
    i2 = jnp.min(jnp.where(l2 == m2, eidx, big), axis=1, keepdims=True)

    ed = jnp.exp(m2 - m1)
    g0 = 1.0 / (1.0 + ed)
    g1 = ed / (1.0 + ed)

    ohc = ((eidx == i1) | (eidx == i2 + E)).astype(jnp.float32)
    cum = ohc
    s = 1
    while s < N:
        shifted = jnp.concatenate(
            [jnp.zeros((s, EPAD), jnp.float32), cum[: N - s, :]], axis=0)
        cum = cum + shifted
        s *= 2
    oh0f = (eidx == i1).astype(jnp.float32)
    oh1f = (eidx == i2 + E).astype(jnp.float32)
    p0 = jnp.sum(cum * oh0f, axis=1, keepdims=True)
    p1 = jnp.sum(cum * oh1f, axis=1, keepdims=True)
    s0 = p0.astype(jnp.int32) - 1
    s1 = p1.astype(jnp.int32) - 1
    v0 = s0 < CAP
    v1 = s1 < CAP
    f0_ref[...] = jnp.where(v0, i1 * CAP + s0, SENT)
    f1_ref[...] = jnp.where(v1, i2 * CAP + s1, SENT)
    w0_ref[...] = jnp.where(v0, g0, 0.0)
    w1_ref[...] = jnp.where(v1, g1, 0.0)


_router = pl.pallas_call(
    _router_body,
    out_shape=(
        jax.ShapeDtypeStruct((N, 1), jnp.int32),
        jax.ShapeDtypeStruct((N, 1), jnp.int32),
        jax.ShapeDtypeStruct((N, 1), jnp.float32),
        jax.ShapeDtypeStruct((N, 1), jnp.float32),
        jax.ShapeDtypeStruct((N, D), jnp.bfloat16),
    ),
)


HBLK = 2048
NHBLK = H // HBLK


def _moe_body(f0r_ref, f1r_ref, xb_ref, w1_ref, w2_ref, out_ref, ei_s):
    e = pl.program_id(0)
    h = pl.program_id(1)

    @pl.when(h == 0)
    def _():
        slot = e * CAP + lax.broadcasted_iota(jnp.int32, (CAP, N), 0)
        oh = ((f0r_ref[...] == slot) | (f1r_ref[...] == slot))
        ei = jnp.dot(oh.astype(jnp.bfloat16), xb_ref[...],
                     preferred_element_type=jnp.float32)
        ei_s[...] = ei.astype(jnp.bfloat16)

    act = jnp.dot(ei_s[...], w1_ref[0].astype(jnp.bfloat16),
                  preferred_element_type=jnp.float32)
    act = jax.nn.gelu(act).astype(jnp.bfloat16)
    part = jnp.dot(act, w2_ref[0].astype(jnp.bfloat16),
                   preferred_element_type=jnp.float32)

    @pl.when(h == 0)
    def _():
        out_ref[...] = part

    @pl.when(h != 0)
    def _():
        out_ref[...] += part


_moe = pl.pallas_call(
    _moe_body,
    grid=(E, NHBLK),
    in_specs=[
        pl.BlockSpec((1, N), lambda e, h: (0, 0)),
        pl.BlockSpec((1, N), lambda e, h: (0, 0)),
        pl.BlockSpec((N, D), lambda e, h: (0, 0)),
        pl.BlockSpec((1, D, HBLK), lambda e, h: (e, 0, h)),
        pl.BlockSpec((1, HBLK, D), lambda e, h: (e, h, 0)),
    ],
    out_specs=pl.BlockSpec((CAP, D), lambda e, h: (e, 0)),
    out_shape=jax.ShapeDtypeStruct((E * CAP, D), jnp.float32),
    scratch_shapes=[pltpu.VMEM((CAP, D), jnp.bfloat16)],
    compiler_params=pltpu.CompilerParams(
        dimension_semantics=("parallel", "arbitrary"),
    ),
)


@functools.cache
def _sc_kernels():
    mesh = plsc.VectorSubcoreMesh(
        core_axis_name="c", subcore_axis_name="s",
        num_cores=NC, num_subcores=NS,
    )

    CC = 16
    NCH = TPW // CC

    @functools.partial(
        pl.kernel,
        out_type=jax.ShapeDtypeStruct((N, D), jnp.float32),
        mesh=mesh,
        compiler_params=pltpu.CompilerParams(needs_layout_passes=False),
        scratch_types=[
            pltpu.VMEM((TPW,), jnp.int32),
            pltpu.VMEM((TPW,), jnp.int32),
            pltpu.VMEM((TPW,), jnp.float32),
            pltpu.VMEM((TPW,), jnp.float32),
            pltpu.VMEM((2, CC, D), jnp.float32),
            pltpu.VMEM((2, CC, D), jnp.float32),
            pltpu.SemaphoreType.DMA,
            pltpu.SemaphoreType.DMA,
            pltpu.SemaphoreType.DMA,
        ],
    )
    def combine(eo_hbm, f0_hbm, f1_hbm, w0_hbm, w1_hbm, out_hbm,
                i0_v, i1_v, w0_v, w1_v, ra_v, rb_v, sem_a, sem_b, sem_w):
        wid = lax.axis_index("s") * NC + lax.axis_index("c")
        lo = wid * TPW

        pltpu.sync_copy(f0_hbm.at[pl.ds(lo, TPW)], i0_v)
        pltpu.sync_copy(f1_hbm.at[pl.ds(lo, TPW)], i1_v)
        pltpu.sync_copy(w0_hbm.at[pl.ds(lo, TPW)], w0_v)
        pltpu.sync_copy(w1_hbm.at[pl.ds(lo, TPW)], w1_v)

        lim = jnp.full((L,), E * CAP - 1, jnp.int32)
        for j in range(TPW // L):
            sl = pl.ds(j * L, L)
            i0_v[sl] = jnp.minimum(i0_v[sl], lim)
            i1_v[sl] = jnp.minimum(i1_v[sl], lim)

        def gathers(ci):
            b = ci % 2
            ga = pltpu.async_copy(
                eo_hbm.at[i0_v.at[pl.ds(ci * CC, CC)]], ra_v.at[b], sem_a)
            gb = pltpu.async_copy(
                eo_hbm.at[i1_v.at[pl.ds(ci * CC, CC)]], rb_v.at[b], sem_b)
            return ga, gb

        inflight = gathers(0)
        writes = [None, None]
        for ci in range(NCH):
            b = ci % 2
            ga, gb = inflight
            ga.wait()
            gb.wait()
            if ci + 1 < NCH:
                ob = (ci + 1) % 2
                if writes[ob] is not None:
                    writes[ob].wait()
                    writes[ob] = None
                inflight = gathers(ci + 1)

            def wsum_row(rr, _):
                bidx = jnp.full((L,), ci * CC, jnp.int32) + rr
                w0s = plsc.load_gather(w0_v, [bidx])
                w1s = plsc.load_gather(w1_v, [bidx])
                for j in range(D // L):
                    sl = pl.ds(j * L, L)
                    ra_v[b, rr, sl] = (
                        ra_v[b, rr, sl] * w0s + rb_v[b, rr, sl] * w1s)
                return 0

            lax.fori_loop(0, CC, wsum_row, 0)
            writes[b] = pltpu.async_copy(
                ra_v.at[b], out_hbm.at[pl.ds(lo + ci * CC, CC)], sem_w)
        for wr in writes:
            if wr is not None:
                wr.wait()

    return combine


def kernel(x, W_hash, expert_w1, expert_w2):
    combine = _sc_kernels()

    x_flat = x.reshape(N, D)
    whp = jnp.pad(W_hash, ((0, 0), (0, EPAD - E)))

    f0, f1, w0, w1, xb = _router(x_flat, whp)
    f0r = f0.reshape(1, N)
    f1r = f1.reshape(1, N)

    eo = _moe(f0r, f1r, xb, expert_w1, expert_w2)
    out = combine(eo, f0.reshape(N), f1.reshape(N), w0.reshape(N), w1.reshape(N))
    return out.reshape(1, N, D)
